# baseline (speedup 1.0000x reference)
import functools

import jax
import jax.numpy as jnp
from jax import lax
from jax.experimental import pallas as pl
from jax.experimental.pallas import tpu as pltpu
from jax.experimental.pallas import tpu_sc as plsc

N_NODES = 10000
N_EDGES = 320000
D = 128
N_CLASSES = 16
N_STEPS = 3

NC = 2
NS = 16
NW = NC * NS
K = 40
G = 8
IDXG = 16
CH_CORE = (368, 144)
TOT_CHUNKS = NS * (CH_CORE[0] + CH_CORE[1])
SPLIT = NS * CH_CORE[0]
EDGES_PAD = TOT_CHUNKS * K
N_ACC = N_NODES + (-N_NODES) % (NS * 8)
ROWS_PER_SUB = N_ACC // NS

_PREC = None


def _make_scatter():
    mesh = plsc.VectorSubcoreMesh(core_axis_name="c", subcore_axis_name="s")

    @functools.partial(
        pl.kernel,
        out_type=jax.ShapeDtypeStruct((NC, N_ACC, D), jnp.float32),
        mesh=mesh,
        scratch_types=(
            [pltpu.VMEM((IDXG, K), jnp.int32)] * 2 +
            [pltpu.VMEM((K, D), jnp.float32)] * G +
            [pltpu.VMEM_SHARED((N_ACC, D), jnp.float32)] +
            [pltpu.SemaphoreType.DMA] * (2 * G)
        ),
    )
    def scatter_kernel(m_hbm, src_hbm, dst_hbm, out_hbm,
                       src_v, dst_v, *rest):
        bufs = rest[:G]
        acc_sh = rest[G]
        gsems = rest[G + 1:2 * G + 1]
        ssems = rest[2 * G + 1:]
        c = lax.axis_index("c")
        s = lax.axis_index("s")
        base = jnp.where(c == 0, s * CH_CORE[0], SPLIT + s * CH_CORE[1])
        nblk = jnp.where(c == 0, CH_CORE[0] // IDXG, CH_CORE[1] // IDXG)
        row0 = s * ROWS_PER_SUB
        def zrow(i, carry):
            for jj in range(D // 16):
                bufs[0][i, pl.ds(jj * 16, 16)] = jnp.zeros((16,), jnp.float32)
            return carry
        lax.fori_loop(0, K, zrow, 0)
        for t in range(ROWS_PER_SUB // K):
            pltpu.sync_copy(bufs[0], acc_sh.at[pl.ds(row0 + t * K, K)])
        rem = ROWS_PER_SUB % K
        if rem:
            pltpu.sync_copy(bufs[0].at[pl.ds(0, rem)],
                            acc_sh.at[pl.ds(row0 + ROWS_PER_SUB - rem, rem)])
        plsc.subcore_barrier()

        pltpu.sync_copy(src_hbm.at[pl.ds(base, IDXG)], src_v)
        pltpu.sync_copy(dst_hbm.at[pl.ds(base, IDXG)], dst_v)
        for r in range(G):
            pltpu.async_copy(m_hbm.at[src_v.at[r]], bufs[r], gsems[r])
        for r in range(G):
            pltpu.make_async_copy(m_hbm.at[src_v.at[r]], bufs[r],
                                  gsems[r]).wait()
            pltpu.async_copy(bufs[r], acc_sh.at[dst_v.at[r]], ssems[r],
                             add=True)
        for r in range(G):
            q = G + r
            pltpu.make_async_copy(bufs[r], acc_sh.at[dst_v.at[q]],
                                  ssems[r]).wait()
            pltpu.async_copy(m_hbm.at[src_v.at[q]], bufs[r], gsems[r])
        for r in range(G):
            q = G + r
            pltpu.make_async_copy(m_hbm.at[src_v.at[q]], bufs[r],
                                  gsems[r]).wait()
            pltpu.async_copy(bufs[r], acc_sh.at[dst_v.at[q]], ssems[r],
                             add=True)

        def body(g, carry):
            blk = base + g * IDXG
            pltpu.sync_copy(src_hbm.at[pl.ds(blk, IDXG)], src_v)
            pltpu.sync_copy(dst_hbm.at[pl.ds(blk, IDXG)], dst_v)
            for half in range(2):
                for r in range(G):
                    q = half * G + r
                    pltpu.make_async_copy(bufs[r], acc_sh.at[dst_v.at[q]],
                                          ssems[r]).wait()
                    pltpu.async_copy(m_hbm.at[src_v.at[q]], bufs[r],
                                     gsems[r])
                for r in range(G):
                    q = half * G + r
                    pltpu.make_async_copy(m_hbm.at[src_v.at[q]], bufs[r],
                                          gsems[r]).wait()
                    pltpu.async_copy(bufs[r], acc_sh.at[dst_v.at[q]],
                                     ssems[r], add=True)
            return carry
        lax.fori_loop(1, nblk, body, 0)
        for r in range(G):
            pltpu.make_async_copy(bufs[r], acc_sh.at[dst_v.at[r]],
                                  ssems[r]).wait()
        plsc.subcore_barrier()
        pltpu.sync_copy(acc_sh.at[pl.ds(row0, ROWS_PER_SUB)],
                        out_hbm.at[c, pl.ds(row0, ROWS_PER_SUB)])

    return scatter_kernel


_scatter = _make_scatter()


_B = 1000
_NBLK = N_NODES // _B


def _dotT(a, w):
    return jax.lax.dot_general(a, w, (((1,), (1,)), ((), ())),
                               precision=_PREC,
                               preferred_element_type=jnp.float32)


def _msg_body(h_ref, w_ref, b_ref, out_ref):
    out_ref[...] = _dotT(h_ref[...], w_ref[...]) + b_ref[...]


def _msg(h, W, Wb):
    return pl.pallas_call(
        _msg_body,
        grid=(_NBLK,),
        in_specs=[
            pl.BlockSpec((_B, D), lambda i: (i, 0)),
            pl.BlockSpec((D, D), lambda i: (0, 0)),
            pl.BlockSpec((1, D), lambda i: (0, 0)),
        ],
        out_specs=pl.BlockSpec((_B, D), lambda i: (i, 0)),
        out_shape=jax.ShapeDtypeStruct((N_NODES, D), jnp.float32),
    )(h, W, Wb)


def _gru_math(a, h, wih, whh, bih, bhh):
    gi = _dotT(a, wih) + bih
    gh = _dotT(h, whh) + bhh
    r = jax.nn.sigmoid(gi[:, :D] + gh[:, :D])
    z = jax.nn.sigmoid(gi[:, D:2 * D] + gh[:, D:2 * D])
    n = jnp.tanh(gi[:, 2 * D:] + r * gh[:, 2 * D:])
    return (1.0 - z) * n + z * h


def _gru_body(a2_ref, h_ref, wih_ref, whh_ref, bih_ref, bhh_ref,
              w_ref, wb_ref, h_out, m_out):
    a = a2_ref[0] + a2_ref[1]
    hn = _gru_math(a, h_ref[...], wih_ref[...], whh_ref[...],
                   bih_ref[...], bhh_ref[...])
    h_out[...] = hn
    m_out[...] = _dotT(hn, w_ref[...]) + wb_ref[...]


def _gru_step(a2, h, Wih, Whh, bih, bhh, Wn, Wbn):
    return pl.pallas_call(
        _gru_body,
        grid=(_NBLK,),
        in_specs=[
            pl.BlockSpec((NC, _B, D), lambda i: (0, i, 0)),
            pl.BlockSpec((_B, D), lambda i: (i, 0)),
            pl.BlockSpec((3 * D, D), lambda i: (0, 0)),
            pl.BlockSpec((3 * D, D), lambda i: (0, 0)),
            pl.BlockSpec((1, 3 * D), lambda i: (0, 0)),
            pl.BlockSpec((1, 3 * D), lambda i: (0, 0)),
            pl.BlockSpec((D, D), lambda i: (0, 0)),
            pl.BlockSpec((1, D), lambda i: (0, 0)),
        ],
        out_specs=[
            pl.BlockSpec((_B, D), lambda i: (i, 0)),
            pl.BlockSpec((_B, D), lambda i: (i, 0)),
        ],
        out_shape=[
            jax.ShapeDtypeStruct((N_NODES, D), jnp.float32),
            jax.ShapeDtypeStruct((N_NODES, D), jnp.float32),
        ],
    )(a2, h, Wih, Whh, bih, bhh, Wn, Wbn)


def _gru_last_body(a2_ref, h_ref, wih_ref, whh_ref, bih_ref, bhh_ref,
                   wc_ref, bc_ref, out_ref, acc_ref):
    @pl.when(pl.program_id(0) == 0)
    def _():
        acc_ref[...] = jnp.zeros_like(acc_ref)

    a = a2_ref[0] + a2_ref[1]
    hn = _gru_math(a, h_ref[...], wih_ref[...], whh_ref[...],
                   bih_ref[...], bhh_ref[...])
    acc_ref[...] += jnp.sum(hn, axis=0, keepdims=True)

    @pl.when(pl.program_id(0) == _NBLK - 1)
    def _():
        hg = acc_ref[...] * (1.0 / N_NODES)
        out_ref[...] = _dotT(hg, wc_ref[...]) + bc_ref[...]


def _gru_last(a2, h, Wih, Whh, bih, bhh, Wc, bc):
    return pl.pallas_call(
        _gru_last_body,
        grid=(_NBLK,),
        in_specs=[
            pl.BlockSpec((NC, _B, D), lambda i: (0, i, 0)),
            pl.BlockSpec((_B, D), lambda i: (i, 0)),
            pl.BlockSpec((3 * D, D), lambda i: (0, 0)),
            pl.BlockSpec((3 * D, D), lambda i: (0, 0)),
            pl.BlockSpec((1, 3 * D), lambda i: (0, 0)),
            pl.BlockSpec((1, 3 * D), lambda i: (0, 0)),
            pl.BlockSpec((N_CLASSES, D), lambda i: (0, 0)),
            pl.BlockSpec((1, N_CLASSES), lambda i: (0, 0)),
        ],
        out_specs=pl.BlockSpec((1, N_CLASSES), lambda i: (0, 0)),
        out_shape=jax.ShapeDtypeStruct((1, N_CLASSES), jnp.float32),
        scratch_shapes=[pltpu.VMEM((1, D), jnp.float32)],
    )(a2, h, Wih, Whh, bih, bhh, Wc, bc)


def kernel(x, edge_index, W0, Wb0, Wih0, Whh0, bih0, bhh0,
           W1, Wb1, Wih1, Whh1, bih1, bhh1, Wc, bc):
    pad = EDGES_PAD - N_EDGES
    srcp = jnp.pad(edge_index[0], (0, pad)).reshape(TOT_CHUNKS, K)
    dstp = jnp.pad(edge_index[1], (0, pad),
                   constant_values=N_NODES).reshape(TOT_CHUNKS, K)
    Wb0r, Wb1r = Wb0.reshape(1, D), Wb1.reshape(1, D)
    bih0r, bhh0r = bih0.reshape(1, 3 * D), bhh0.reshape(1, 3 * D)
    bih1r, bhh1r = bih1.reshape(1, 3 * D), bhh1.reshape(1, 3 * D)
    bcr = bc.reshape(1, N_CLASSES)

    h = x
    m = _msg(h, W0, Wb0r)
    layers = [(Wih0, Whh0, bih0r, bhh0r), (Wih1, Whh1, bih1r, bhh1r)]
    for l in range(2):
        Wih, Whh, bihr, bhhr = layers[l]
        for step in range(N_STEPS):
            a2 = _scatter(m, srcp, dstp)
            last_of_layer = step == N_STEPS - 1
            if l == 1 and last_of_layer:
                return _gru_last(a2, h, Wih, Whh, bihr, bhhr, Wc, bcr)
            Wn, Wbn = (W0, Wb0r) if (l == 0 and not last_of_layer) else (W1, Wb1r)
            h, m = _gru_step(a2, h, Wih, Whh, bihr, bhhr, Wn, Wbn)


# speedup vs baseline: 1.8579x; 1.8579x over previous
import functools

import jax
import jax.numpy as jnp
from jax import lax
from jax.experimental import pallas as pl
from jax.experimental.pallas import tpu as pltpu
from jax.experimental.pallas import tpu_sc as plsc

N_NODES = 10000
N_EDGES = 320000
D = 128
N_CLASSES = 16
N_STEPS = 3

NC = 2
NS = 16
NW = NC * NS
K = 40
G = 8
CH_CORE = (368, 136)
TOT_CHUNKS = NS * (CH_CORE[0] + CH_CORE[1])
SPLIT = NS * CH_CORE[0]
EDGES_PAD = TOT_CHUNKS * K
N_ACC = N_NODES + (-N_NODES) % (NS * 8)
ROWS_PER_SUB = N_ACC // NS

_PREC = None


def _make_scatter():
    mesh = plsc.VectorSubcoreMesh(core_axis_name="c", subcore_axis_name="s")

    @functools.partial(
        pl.kernel,
        out_type=jax.ShapeDtypeStruct((NC, N_ACC, D), jnp.float32),
        mesh=mesh,
        scratch_types=(
            [pltpu.VMEM((G, K), jnp.int32)] * 2 +
            [pltpu.VMEM((K, D), jnp.float32)] * G +
            [pltpu.VMEM_SHARED((N_ACC, D), jnp.float32)] +
            [pltpu.SemaphoreType.DMA] * (2 * G)
        ),
    )
    def scatter_kernel(m_hbm, src_hbm, dst_hbm, out_hbm,
                       src_v, dst_v, *rest):
        bufs = rest[:G]
        acc_sh = rest[G]
        gsems = rest[G + 1:2 * G + 1]
        ssems = rest[2 * G + 1:]
        c = lax.axis_index("c")
        s = lax.axis_index("s")
        base = jnp.where(c == 0, s * CH_CORE[0], SPLIT + s * CH_CORE[1])
        nblk = jnp.where(c == 0, CH_CORE[0] // G, CH_CORE[1] // G)
        row0 = s * ROWS_PER_SUB
        def zrow(i, carry):
            for jj in range(D // 16):
                bufs[0][i, pl.ds(jj * 16, 16)] = jnp.zeros((16,), jnp.float32)
            return carry
        lax.fori_loop(0, K, zrow, 0)
        for t in range(ROWS_PER_SUB // K):
            pltpu.sync_copy(bufs[0], acc_sh.at[pl.ds(row0 + t * K, K)])
        rem = ROWS_PER_SUB % K
        if rem:
            pltpu.sync_copy(bufs[0].at[pl.ds(0, rem)],
                            acc_sh.at[pl.ds(row0 + ROWS_PER_SUB - rem, rem)])
        plsc.subcore_barrier()

        pltpu.sync_copy(src_hbm.at[pl.ds(base, G)], src_v)
        pltpu.sync_copy(dst_hbm.at[pl.ds(base, G)], dst_v)
        for r in range(G):
            pltpu.async_copy(m_hbm.at[src_v.at[r]], bufs[r], gsems[r])
        for r in range(G):
            pltpu.make_async_copy(m_hbm.at[src_v.at[r]], bufs[r],
                                  gsems[r]).wait()
            pltpu.async_copy(bufs[r], acc_sh.at[dst_v.at[r]], ssems[r],
                             add=True)

        def body(g, carry):
            blk = base + g * G
            pltpu.sync_copy(src_hbm.at[pl.ds(blk, G)], src_v)
            pltpu.sync_copy(dst_hbm.at[pl.ds(blk, G)], dst_v)
            for r in range(G):
                pltpu.make_async_copy(bufs[r], acc_sh.at[dst_v.at[r]],
                                      ssems[r]).wait()
                pltpu.async_copy(m_hbm.at[src_v.at[r]], bufs[r], gsems[r])
            for r in range(G):
                pltpu.make_async_copy(m_hbm.at[src_v.at[r]], bufs[r],
                                      gsems[r]).wait()
                pltpu.async_copy(bufs[r], acc_sh.at[dst_v.at[r]], ssems[r],
                                 add=True)
            return carry
        lax.fori_loop(1, nblk, body, 0)
        for r in range(G):
            pltpu.make_async_copy(bufs[r], acc_sh.at[dst_v.at[r]],
                                  ssems[r]).wait()
        plsc.subcore_barrier()
        pltpu.sync_copy(acc_sh.at[pl.ds(row0, ROWS_PER_SUB)],
                        out_hbm.at[c, pl.ds(row0, ROWS_PER_SUB)])

    return scatter_kernel


_scatter = _make_scatter()


_B = 1000
_NBLK = N_NODES // _B


def _dotT(a, w):
    return jax.lax.dot_general(a, w, (((1,), (1,)), ((), ())),
                               precision=_PREC,
                               preferred_element_type=jnp.float32)


def _msg_body(h_ref, w_ref, b_ref, out_ref):
    out_ref[...] = _dotT(h_ref[...], w_ref[...]) + b_ref[...]


def _msg(h, W, Wb):
    return pl.pallas_call(
        _msg_body,
        grid=(_NBLK,),
        in_specs=[
            pl.BlockSpec((_B, D), lambda i: (i, 0)),
            pl.BlockSpec((D, D), lambda i: (0, 0)),
            pl.BlockSpec((1, D), lambda i: (0, 0)),
        ],
        out_specs=pl.BlockSpec((_B, D), lambda i: (i, 0)),
        out_shape=jax.ShapeDtypeStruct((N_NODES, D), jnp.float32),
    )(h, W, Wb)


def _gru_math(a, h, wih, whh, bih, bhh):
    gi = _dotT(a, wih) + bih
    gh = _dotT(h, whh) + bhh
    r = jax.nn.sigmoid(gi[:, :D] + gh[:, :D])
    z = jax.nn.sigmoid(gi[:, D:2 * D] + gh[:, D:2 * D])
    n = jnp.tanh(gi[:, 2 * D:] + r * gh[:, 2 * D:])
    return (1.0 - z) * n + z * h


def _gru_body(a2_ref, h_ref, wih_ref, whh_ref, bih_ref, bhh_ref,
              w_ref, wb_ref, h_out, m_out):
    a = a2_ref[0] + a2_ref[1]
    hn = _gru_math(a, h_ref[...], wih_ref[...], whh_ref[...],
                   bih_ref[...], bhh_ref[...])
    h_out[...] = hn
    m_out[...] = _dotT(hn, w_ref[...]) + wb_ref[...]


def _gru_step(a2, h, Wih, Whh, bih, bhh, Wn, Wbn):
    return pl.pallas_call(
        _gru_body,
        grid=(_NBLK,),
        in_specs=[
            pl.BlockSpec((NC, _B, D), lambda i: (0, i, 0)),
            pl.BlockSpec((_B, D), lambda i: (i, 0)),
            pl.BlockSpec((3 * D, D), lambda i: (0, 0)),
            pl.BlockSpec((3 * D, D), lambda i: (0, 0)),
            pl.BlockSpec((1, 3 * D), lambda i: (0, 0)),
            pl.BlockSpec((1, 3 * D), lambda i: (0, 0)),
            pl.BlockSpec((D, D), lambda i: (0, 0)),
            pl.BlockSpec((1, D), lambda i: (0, 0)),
        ],
        out_specs=[
            pl.BlockSpec((_B, D), lambda i: (i, 0)),
            pl.BlockSpec((_B, D), lambda i: (i, 0)),
        ],
        out_shape=[
            jax.ShapeDtypeStruct((N_NODES, D), jnp.float32),
            jax.ShapeDtypeStruct((N_NODES, D), jnp.float32),
        ],
    )(a2, h, Wih, Whh, bih, bhh, Wn, Wbn)


def _gru_last_body(a2_ref, h_ref, wih_ref, whh_ref, bih_ref, bhh_ref,
                   wc_ref, bc_ref, out_ref, acc_ref):
    @pl.when(pl.program_id(0) == 0)
    def _():
        acc_ref[...] = jnp.zeros_like(acc_ref)

    a = a2_ref[0] + a2_ref[1]
    hn = _gru_math(a, h_ref[...], wih_ref[...], whh_ref[...],
                   bih_ref[...], bhh_ref[...])
    acc_ref[...] += jnp.sum(hn, axis=0, keepdims=True)

    @pl.when(pl.program_id(0) == _NBLK - 1)
    def _():
        hg = acc_ref[...] * (1.0 / N_NODES)
        out_ref[...] = _dotT(hg, wc_ref[...]) + bc_ref[...]


def _gru_last(a2, h, Wih, Whh, bih, bhh, Wc, bc):
    return pl.pallas_call(
        _gru_last_body,
        grid=(_NBLK,),
        in_specs=[
            pl.BlockSpec((NC, _B, D), lambda i: (0, i, 0)),
            pl.BlockSpec((_B, D), lambda i: (i, 0)),
            pl.BlockSpec((3 * D, D), lambda i: (0, 0)),
            pl.BlockSpec((3 * D, D), lambda i: (0, 0)),
            pl.BlockSpec((1, 3 * D), lambda i: (0, 0)),
            pl.BlockSpec((1, 3 * D), lambda i: (0, 0)),
            pl.BlockSpec((N_CLASSES, D), lambda i: (0, 0)),
            pl.BlockSpec((1, N_CLASSES), lambda i: (0, 0)),
        ],
        out_specs=pl.BlockSpec((1, N_CLASSES), lambda i: (0, 0)),
        out_shape=jax.ShapeDtypeStruct((1, N_CLASSES), jnp.float32),
        scratch_shapes=[pltpu.VMEM((1, D), jnp.float32)],
    )(a2, h, Wih, Whh, bih, bhh, Wc, bc)


def kernel(x, edge_index, W0, Wb0, Wih0, Whh0, bih0, bhh0,
           W1, Wb1, Wih1, Whh1, bih1, bhh1, Wc, bc):
    pad = EDGES_PAD - N_EDGES
    srcp = jnp.pad(edge_index[0], (0, pad)).reshape(TOT_CHUNKS, K)
    dstp = jnp.pad(edge_index[1], (0, pad),
                   constant_values=N_NODES).reshape(TOT_CHUNKS, K)
    Wb0r, Wb1r = Wb0.reshape(1, D), Wb1.reshape(1, D)
    bih0r, bhh0r = bih0.reshape(1, 3 * D), bhh0.reshape(1, 3 * D)
    bih1r, bhh1r = bih1.reshape(1, 3 * D), bhh1.reshape(1, 3 * D)
    bcr = bc.reshape(1, N_CLASSES)

    h = x
    m = _msg(h, W0, Wb0r)
    layers = [(Wih0, Whh0, bih0r, bhh0r), (Wih1, Whh1, bih1r, bhh1r)]
    for l in range(2):
        Wih, Whh, bihr, bhhr = layers[l]
        for step in range(N_STEPS):
            a2 = _scatter(m, srcp, dstp)
            last_of_layer = step == N_STEPS - 1
            if l == 1 and last_of_layer:
                return _gru_last(a2, h, Wih, Whh, bihr, bhhr, Wc, bcr)
            Wn, Wbn = (W0, Wb0r) if (l == 0 and not last_of_layer) else (W1, Wb1r)
            h, m = _gru_step(a2, h, Wih, Whh, bihr, bhhr, Wn, Wbn)


# speedup vs baseline: 1.8806x; 1.0122x over previous
import functools

import jax
import jax.numpy as jnp
from jax import lax
from jax.experimental import pallas as pl
from jax.experimental.pallas import tpu as pltpu
from jax.experimental.pallas import tpu_sc as plsc

N_NODES = 10000
N_EDGES = 320000
D = 128
N_CLASSES = 16
N_STEPS = 3

NC = 2
NS = 16
NW = NC * NS
K = 40
G = 8
CH_CORE = (368, 136)
TOT_CHUNKS = NS * (CH_CORE[0] + CH_CORE[1])
SPLIT = NS * CH_CORE[0]
EDGES_PAD = TOT_CHUNKS * K
N_ACC = N_NODES + (-N_NODES) % (NS * 8)
ROWS_PER_SUB = N_ACC // NS

_PREC = None


def _make_scatter():
    mesh = plsc.VectorSubcoreMesh(core_axis_name="c", subcore_axis_name="s")

    @functools.partial(
        pl.kernel,
        out_type=jax.ShapeDtypeStruct((NC, N_ACC, D), jnp.float32),
        mesh=mesh,
        scratch_types=(
            [pltpu.VMEM((G, K), jnp.int32)] * 2 +
            [pltpu.VMEM((K, D), jnp.float32)] * G +
            [pltpu.VMEM_SHARED((N_ACC, D), jnp.float32)] +
            [pltpu.SemaphoreType.DMA] * (2 * G)
        ),
    )
    def scatter_kernel(m_hbm, src_hbm, dst_hbm, out_hbm,
                       src_v, dst_v, *rest):
        bufs = rest[:G]
        acc_sh = rest[G]
        gsems = rest[G + 1:2 * G + 1]
        ssems = rest[2 * G + 1:]
        c = lax.axis_index("c")
        s = lax.axis_index("s")
        base = jnp.where(c == 0, s * CH_CORE[0], SPLIT + s * CH_CORE[1])
        nblk = jnp.where(c == 0, CH_CORE[0] // G, CH_CORE[1] // G)
        row0 = s * ROWS_PER_SUB
        def zrow(i, carry):
            for jj in range(D // 16):
                bufs[0][i, pl.ds(jj * 16, 16)] = jnp.zeros((16,), jnp.float32)
            return carry
        lax.fori_loop(0, K, zrow, 0)
        for t in range(ROWS_PER_SUB // K):
            pltpu.sync_copy(bufs[0], acc_sh.at[pl.ds(row0 + t * K, K)])
        rem = ROWS_PER_SUB % K
        if rem:
            pltpu.sync_copy(bufs[0].at[pl.ds(0, rem)],
                            acc_sh.at[pl.ds(row0 + ROWS_PER_SUB - rem, rem)])
        plsc.subcore_barrier()

        pltpu.sync_copy(src_hbm.at[pl.ds(base, G)], src_v)
        pltpu.sync_copy(dst_hbm.at[pl.ds(base, G)], dst_v)
        for r in range(G):
            pltpu.async_copy(m_hbm.at[src_v.at[r]], bufs[r], gsems[r])
        for r in range(G):
            pltpu.make_async_copy(m_hbm.at[src_v.at[r]], bufs[r],
                                  gsems[r]).wait()
            pltpu.async_copy(bufs[r], acc_sh.at[dst_v.at[r]], ssems[r],
                             add=True)

        def body(g, carry):
            blk = base + g * G
            pltpu.sync_copy(src_hbm.at[pl.ds(blk, G)], src_v)
            pltpu.sync_copy(dst_hbm.at[pl.ds(blk, G)], dst_v)
            for r in range(G):
                pltpu.make_async_copy(bufs[r], acc_sh.at[dst_v.at[r]],
                                      ssems[r]).wait()
                pltpu.async_copy(m_hbm.at[src_v.at[r]], bufs[r], gsems[r])
            for r in range(G):
                pltpu.make_async_copy(m_hbm.at[src_v.at[r]], bufs[r],
                                      gsems[r]).wait()
                pltpu.async_copy(bufs[r], acc_sh.at[dst_v.at[r]], ssems[r],
                                 add=True)
            return carry
        lax.fori_loop(1, nblk, body, 0)
        for r in range(G):
            pltpu.make_async_copy(bufs[r], acc_sh.at[dst_v.at[r]],
                                  ssems[r]).wait()
        plsc.subcore_barrier()
        pltpu.sync_copy(acc_sh.at[pl.ds(row0, ROWS_PER_SUB)],
                        out_hbm.at[c, pl.ds(row0, ROWS_PER_SUB)])

    return scatter_kernel


_scatter = _make_scatter()


_B = 2000
_NBLK = N_NODES // _B


def _dotT(a, w):
    return jax.lax.dot_general(a, w, (((1,), (1,)), ((), ())),
                               precision=_PREC,
                               preferred_element_type=jnp.float32)


def _msg_body(h_ref, w_ref, b_ref, out_ref):
    out_ref[...] = _dotT(h_ref[...], w_ref[...]) + b_ref[...]


def _msg(h, W, Wb):
    return pl.pallas_call(
        _msg_body,
        grid=(_NBLK,),
        in_specs=[
            pl.BlockSpec((_B, D), lambda i: (i, 0)),
            pl.BlockSpec((D, D), lambda i: (0, 0)),
            pl.BlockSpec((1, D), lambda i: (0, 0)),
        ],
        out_specs=pl.BlockSpec((_B, D), lambda i: (i, 0)),
        out_shape=jax.ShapeDtypeStruct((N_NODES, D), jnp.float32),
    )(h, W, Wb)


def _gru_math(a, h, wih, whh, bih, bhh):
    gi = _dotT(a, wih) + bih
    gh = _dotT(h, whh) + bhh
    r = jax.nn.sigmoid(gi[:, :D] + gh[:, :D])
    z = jax.nn.sigmoid(gi[:, D:2 * D] + gh[:, D:2 * D])
    n = jnp.tanh(gi[:, 2 * D:] + r * gh[:, 2 * D:])
    return (1.0 - z) * n + z * h


def _gru_body(a2_ref, h_ref, wih_ref, whh_ref, bih_ref, bhh_ref,
              w_ref, wb_ref, h_out, m_out):
    a = a2_ref[0] + a2_ref[1]
    hn = _gru_math(a, h_ref[...], wih_ref[...], whh_ref[...],
                   bih_ref[...], bhh_ref[...])
    h_out[...] = hn
    m_out[...] = _dotT(hn, w_ref[...]) + wb_ref[...]


def _gru_step(a2, h, Wih, Whh, bih, bhh, Wn, Wbn):
    return pl.pallas_call(
        _gru_body,
        grid=(_NBLK,),
        in_specs=[
            pl.BlockSpec((NC, _B, D), lambda i: (0, i, 0)),
            pl.BlockSpec((_B, D), lambda i: (i, 0)),
            pl.BlockSpec((3 * D, D), lambda i: (0, 0)),
            pl.BlockSpec((3 * D, D), lambda i: (0, 0)),
            pl.BlockSpec((1, 3 * D), lambda i: (0, 0)),
            pl.BlockSpec((1, 3 * D), lambda i: (0, 0)),
            pl.BlockSpec((D, D), lambda i: (0, 0)),
            pl.BlockSpec((1, D), lambda i: (0, 0)),
        ],
        out_specs=[
            pl.BlockSpec((_B, D), lambda i: (i, 0)),
            pl.BlockSpec((_B, D), lambda i: (i, 0)),
        ],
        out_shape=[
            jax.ShapeDtypeStruct((N_NODES, D), jnp.float32),
            jax.ShapeDtypeStruct((N_NODES, D), jnp.float32),
        ],
    )(a2, h, Wih, Whh, bih, bhh, Wn, Wbn)


def _gru_last_body(a2_ref, h_ref, wih_ref, whh_ref, bih_ref, bhh_ref,
                   wc_ref, bc_ref, out_ref, acc_ref):
    @pl.when(pl.program_id(0) == 0)
    def _():
        acc_ref[...] = jnp.zeros_like(acc_ref)

    a = a2_ref[0] + a2_ref[1]
    hn = _gru_math(a, h_ref[...], wih_ref[...], whh_ref[...],
                   bih_ref[...], bhh_ref[...])
    acc_ref[...] += jnp.sum(hn, axis=0, keepdims=True)

    @pl.when(pl.program_id(0) == _NBLK - 1)
    def _():
        hg = acc_ref[...] * (1.0 / N_NODES)
        out_ref[...] = _dotT(hg, wc_ref[...]) + bc_ref[...]


def _gru_last(a2, h, Wih, Whh, bih, bhh, Wc, bc):
    return pl.pallas_call(
        _gru_last_body,
        grid=(_NBLK,),
        in_specs=[
            pl.BlockSpec((NC, _B, D), lambda i: (0, i, 0)),
            pl.BlockSpec((_B, D), lambda i: (i, 0)),
            pl.BlockSpec((3 * D, D), lambda i: (0, 0)),
            pl.BlockSpec((3 * D, D), lambda i: (0, 0)),
            pl.BlockSpec((1, 3 * D), lambda i: (0, 0)),
            pl.BlockSpec((1, 3 * D), lambda i: (0, 0)),
            pl.BlockSpec((N_CLASSES, D), lambda i: (0, 0)),
            pl.BlockSpec((1, N_CLASSES), lambda i: (0, 0)),
        ],
        out_specs=pl.BlockSpec((1, N_CLASSES), lambda i: (0, 0)),
        out_shape=jax.ShapeDtypeStruct((1, N_CLASSES), jnp.float32),
        scratch_shapes=[pltpu.VMEM((1, D), jnp.float32)],
    )(a2, h, Wih, Whh, bih, bhh, Wc, bc)


def kernel(x, edge_index, W0, Wb0, Wih0, Whh0, bih0, bhh0,
           W1, Wb1, Wih1, Whh1, bih1, bhh1, Wc, bc):
    pad = EDGES_PAD - N_EDGES
    srcp = jnp.pad(edge_index[0], (0, pad)).reshape(TOT_CHUNKS, K)
    dstp = jnp.pad(edge_index[1], (0, pad),
                   constant_values=N_NODES).reshape(TOT_CHUNKS, K)
    Wb0r, Wb1r = Wb0.reshape(1, D), Wb1.reshape(1, D)
    bih0r, bhh0r = bih0.reshape(1, 3 * D), bhh0.reshape(1, 3 * D)
    bih1r, bhh1r = bih1.reshape(1, 3 * D), bhh1.reshape(1, 3 * D)
    bcr = bc.reshape(1, N_CLASSES)

    h = x
    m = _msg(h, W0, Wb0r)
    layers = [(Wih0, Whh0, bih0r, bhh0r), (Wih1, Whh1, bih1r, bhh1r)]
    for l in range(2):
        Wih, Whh, bihr, bhhr = layers[l]
        for step in range(N_STEPS):
            a2 = _scatter(m, srcp, dstp)
            last_of_layer = step == N_STEPS - 1
            if l == 1 and last_of_layer:
                return _gru_last(a2, h, Wih, Whh, bihr, bhhr, Wc, bcr)
            Wn, Wbn = (W0, Wb0r) if (l == 0 and not last_of_layer) else (W1, Wb1r)
            h, m = _gru_step(a2, h, Wih, Whh, bihr, bhhr, Wn, Wbn)
